# baseline (speedup 1.0000x reference)
import functools

import jax
import jax.numpy as jnp
from jax import lax
from jax.experimental import pallas as pl
from jax.experimental.pallas import tpu as pltpu
from jax.experimental.pallas import tpu_sc as plsc

_BATCH = 4096
_MAXLEN = 200
_EMBED = 64
_NCORES = 2
_NSUB = 16
_NW = _NCORES * _NSUB
_SEQS_W = _BATCH // _NW
_K = 2
_CHUNKS = _SEQS_W // _K
_SUBLEN = 100
_SUB = (_K * _MAXLEN) // _SUBLEN
_XROWS = (_BATCH * _MAXLEN) // _SUBLEN


def _body(x_hbm, tok_hbm, pos_hbm, out_hbm, pos_v, idx_v, rows_v, sem):
    wid = lax.axis_index("s") * _NCORES + lax.axis_index("c")

    pltpu.sync_copy(pos_hbm, pos_v)

    def chunk_body(c, _):
        gc = wid * _CHUNKS + c
        pltpu.sync_copy(x_hbm.at[pl.ds(gc * _SUB, _SUB)], idx_v)
        cps = [
            pltpu.async_copy(tok_hbm.at[idx_v.at[j]], rows_v.at[j], sem)
            for j in range(_SUB)
        ]
        for cp in cps:
            cp.wait()

        def add_body(i, _):
            for h in range(2):
                p = h * _SUBLEN + i
                pvecs = [pos_v[p, pl.ds(16 * jj, 16)] for jj in range(4)]
                for s in range(_K):
                    j = s * 2 + h
                    for jj in range(4):
                        sl = pl.ds(16 * jj, 16)
                        rows_v[j, i, sl] = rows_v[j, i, sl] + pvecs[jj]
            return 0

        lax.fori_loop(0, _SUBLEN, add_body, 0)

        pltpu.sync_copy(rows_v, out_hbm.at[pl.ds(gc * _SUB, _SUB)])
        return 0

    lax.fori_loop(0, _CHUNKS, chunk_body, 0)


_mesh = plsc.VectorSubcoreMesh(core_axis_name="c", subcore_axis_name="s")

_embed = pl.kernel(
    _body,
    out_type=jax.ShapeDtypeStruct((_XROWS, _SUBLEN, _EMBED), jnp.float32),
    mesh=_mesh,
    scratch_types=[
        pltpu.VMEM((_MAXLEN, _EMBED), jnp.float32),
        pltpu.VMEM((_SUB, _SUBLEN), jnp.int32),
        pltpu.VMEM((_SUB, _SUBLEN, _EMBED), jnp.float32),
        pltpu.SemaphoreType.DMA,
    ],
    compiler_params=pltpu.CompilerParams(use_tc_tiling_on_sc=False),
)


@jax.jit
def kernel(x, token_table, pos_table):
    x2 = x.astype(jnp.int32).reshape(_XROWS, _SUBLEN)
    out = _embed(x2, token_table, pos_table)
    return out.reshape(_BATCH, _MAXLEN, _EMBED)
